# baseline (device time: 37711 ns/iter reference)
import jax
import jax.numpy as jnp
from jax import lax
from jax.experimental import pallas as pl
from jax.experimental.pallas import tpu as pltpu

T = 1024
V_SHARD = 8192
D = 1024
TPX = T // 2

C = 8
CH = TPX // C


def kernel(ids, E):
    my_x = lax.axis_index("x")
    my_y = lax.axis_index("y")

    local_ids = lax.dynamic_slice(ids, (my_x * TPX,), (TPX,))
    rel = local_ids - my_y * V_SHARD
    in_range = (rel >= 0) & (rel < V_SHARD)
    idx = jnp.clip(rel, 0, V_SHARD - 1)
    mask = in_range.astype(jnp.float32)[:, None]

    def body(idx_sref, E_ref, m_ref, out_ref,
             gbuf, pbuf, bufy, gsem, ysend, yrecv, xsend, xrecv):
        x = lax.axis_index("x")
        y = lax.axis_index("y")

        def row_copy(c, i):
            r = c * CH + i
            return pltpu.make_async_copy(
                E_ref.at[pl.ds(idx_sref[r], 1), :],
                gbuf.at[pl.ds(r, 1), :],
                gsem.at[c],
            )

        def gather_chunk(c):
            lax.fori_loop(0, CH, lambda i, v: (row_copy(c, i).start(), v)[1], 0)

        def wait_gather_chunk(c):
            pltpu.make_async_copy(
                E_ref.at[pl.ds(0, CH), :],
                gbuf.at[pl.ds(c * CH, CH), :],
                gsem.at[c],
            ).wait()

        gather_chunk(0)

        barrier = pltpu.get_barrier_semaphore()
        pl.semaphore_signal(barrier, inc=1, device_id=(x, 1 - y),
                            device_id_type=pl.DeviceIdType.MESH)
        pl.semaphore_signal(barrier, inc=1, device_id=(1 - x, y),
                            device_id_type=pl.DeviceIdType.MESH)
        pl.semaphore_wait(barrier, 2)

        rdma_ys = []
        for c in range(C):
            wait_gather_chunk(c)
            sl = pl.ds(c * CH, CH)
            pbuf[sl, :] = (gbuf[sl, :] * m_ref[sl, :]).astype(jnp.bfloat16)
            r = pltpu.make_async_remote_copy(
                src_ref=pbuf.at[sl, :],
                dst_ref=bufy.at[sl, :],
                send_sem=ysend.at[c],
                recv_sem=yrecv.at[c],
                device_id=(x, 1 - y),
                device_id_type=pl.DeviceIdType.MESH,
            )
            r.start()
            rdma_ys.append(r)
            if c + 1 < C:
                gather_chunk(c + 1)

        rdma_xs = []
        for c in range(C):
            rdma_ys[c].wait_recv()
            sl = pl.ds(c * CH, CH)
            row0 = x * TPX + c * CH
            out_ref[pl.ds(row0, CH), :] = pbuf[sl, :] + bufy[sl, :]
            r = pltpu.make_async_remote_copy(
                src_ref=out_ref.at[pl.ds(row0, CH), :],
                dst_ref=out_ref.at[pl.ds(row0, CH), :],
                send_sem=xsend.at[c],
                recv_sem=xrecv.at[c],
                device_id=(1 - x, y),
                device_id_type=pl.DeviceIdType.MESH,
            )
            r.start()
            rdma_xs.append(r)

        for c in range(C):
            rdma_ys[c].wait_send()
            rdma_xs[c].wait_send()
            rdma_xs[c].wait_recv()

    grid_spec = pltpu.PrefetchScalarGridSpec(
        num_scalar_prefetch=1,
        in_specs=[
            pl.BlockSpec(memory_space=pl.ANY),
            pl.BlockSpec(memory_space=pltpu.VMEM),
        ],
        out_specs=pl.BlockSpec(memory_space=pltpu.VMEM),
        scratch_shapes=[
            pltpu.VMEM((TPX, D), jnp.float32),
            pltpu.VMEM((TPX, D), jnp.bfloat16),
            pltpu.VMEM((TPX, D), jnp.bfloat16),
            pltpu.SemaphoreType.DMA((C,)),
            pltpu.SemaphoreType.DMA((C,)),
            pltpu.SemaphoreType.DMA((C,)),
            pltpu.SemaphoreType.DMA((C,)),
            pltpu.SemaphoreType.DMA((C,)),
        ],
    )
    return pl.pallas_call(
        body,
        out_shape=jax.ShapeDtypeStruct((T, D), jnp.bfloat16),
        grid_spec=grid_spec,
        compiler_params=pltpu.CompilerParams(collective_id=0),
    )(idx, E, mask)


# device time: 34540 ns/iter; 1.0918x vs baseline; 1.0918x over previous
import jax
import jax.numpy as jnp
from jax import lax
from jax.experimental import pallas as pl
from jax.experimental.pallas import tpu as pltpu

T = 1024
V_SHARD = 8192
D = 1024
TPX = T // 2

C = 4
CH = TPX // C


def kernel(ids, E):
    my_x = lax.axis_index("x")
    my_y = lax.axis_index("y")

    local_ids = lax.dynamic_slice(ids, (my_x * TPX,), (TPX,))
    rel = local_ids - my_y * V_SHARD
    in_range = (rel >= 0) & (rel < V_SHARD)
    idx = jnp.clip(rel, 0, V_SHARD - 1)
    mask = in_range.astype(jnp.float32)[:, None]

    def body(idx_sref, E_ref, m_ref, out_ref,
             gbuf, pbuf, bufy, gsem, ysend, yrecv, xsend, xrecv):
        x = lax.axis_index("x")
        y = lax.axis_index("y")

        for c in range(C):
            def issue(i, v, c=c):
                r = c * CH + i
                pltpu.make_async_copy(
                    E_ref.at[pl.ds(idx_sref[r], 1), :],
                    gbuf.at[pl.ds(r, 1), :],
                    gsem.at[c],
                ).start()
                return v
            lax.fori_loop(0, CH, issue, 0)

        barrier = pltpu.get_barrier_semaphore()
        pl.semaphore_signal(barrier, inc=1, device_id=(x, 1 - y),
                            device_id_type=pl.DeviceIdType.MESH)
        pl.semaphore_signal(barrier, inc=1, device_id=(1 - x, y),
                            device_id_type=pl.DeviceIdType.MESH)
        pl.semaphore_wait(barrier, 2)

        rdma_ys = [None] * C
        rdma_xs = [None] * C

        def y_step(c):
            pltpu.make_async_copy(
                E_ref.at[pl.ds(0, CH), :],
                gbuf.at[pl.ds(c * CH, CH), :],
                gsem.at[c],
            ).wait()
            sl = pl.ds(c * CH, CH)
            pbuf[sl, :] = (gbuf[sl, :] * m_ref[sl, :]).astype(jnp.bfloat16)
            r = pltpu.make_async_remote_copy(
                src_ref=pbuf.at[sl, :],
                dst_ref=bufy.at[sl, :],
                send_sem=ysend.at[c],
                recv_sem=yrecv.at[c],
                device_id=(x, 1 - y),
                device_id_type=pl.DeviceIdType.MESH,
            )
            r.start()
            rdma_ys[c] = r

        def x_step(c):
            rdma_ys[c].wait_recv()
            sl = pl.ds(c * CH, CH)
            row0 = x * TPX + c * CH
            out_ref[pl.ds(row0, CH), :] = pbuf[sl, :] + bufy[sl, :]
            r = pltpu.make_async_remote_copy(
                src_ref=out_ref.at[pl.ds(row0, CH), :],
                dst_ref=out_ref.at[pl.ds(row0, CH), :],
                send_sem=xsend.at[c],
                recv_sem=xrecv.at[c],
                device_id=(1 - x, y),
                device_id_type=pl.DeviceIdType.MESH,
            )
            r.start()
            rdma_xs[c] = r

        for c in range(C):
            y_step(c)
            if c >= 1:
                x_step(c - 1)
        x_step(C - 1)

        for c in range(C):
            rdma_ys[c].wait_send()
            rdma_xs[c].wait_send()
            rdma_xs[c].wait_recv()

    grid_spec = pltpu.PrefetchScalarGridSpec(
        num_scalar_prefetch=1,
        in_specs=[
            pl.BlockSpec(memory_space=pl.ANY),
            pl.BlockSpec(memory_space=pltpu.VMEM),
        ],
        out_specs=pl.BlockSpec(memory_space=pltpu.VMEM),
        scratch_shapes=[
            pltpu.VMEM((TPX, D), jnp.float32),
            pltpu.VMEM((TPX, D), jnp.bfloat16),
            pltpu.VMEM((TPX, D), jnp.bfloat16),
            pltpu.SemaphoreType.DMA((C,)),
            pltpu.SemaphoreType.DMA((C,)),
            pltpu.SemaphoreType.DMA((C,)),
            pltpu.SemaphoreType.DMA((C,)),
            pltpu.SemaphoreType.DMA((C,)),
        ],
    )
    return pl.pallas_call(
        body,
        out_shape=jax.ShapeDtypeStruct((T, D), jnp.bfloat16),
        grid_spec=grid_spec,
        compiler_params=pltpu.CompilerParams(collective_id=0),
    )(idx, E, mask)


# device time: 30341 ns/iter; 1.2429x vs baseline; 1.1384x over previous
import jax
import jax.numpy as jnp
from jax import lax
from jax.experimental import pallas as pl
from jax.experimental.pallas import tpu as pltpu

T = 1024
V_SHARD = 8192
D = 1024
TPX = T // 2

C = 4
CH = TPX // C


def kernel(ids, E):
    my_x = lax.axis_index("x")
    my_y = lax.axis_index("y")

    local_ids = lax.dynamic_slice(ids, (my_x * TPX,), (TPX,))
    rel = local_ids - my_y * V_SHARD
    in_range = (rel >= 0) & (rel < V_SHARD)
    idx = jnp.clip(rel, 0, V_SHARD - 1)
    mask = in_range.astype(jnp.float32)[:, None]

    def body(idx_sref, E_ref, m_ref, out_ref,
             gbuf, pbuf, bufy, gsem, ysend, yrecv, xsend, xrecv):
        x = lax.axis_index("x")
        y = lax.axis_index("y")

        def gather_chunk(c):
            def issue(i, v):
                r = c * CH + i
                pltpu.make_async_copy(
                    E_ref.at[pl.ds(idx_sref[r], 1), :],
                    gbuf.at[pl.ds(r, 1), :],
                    gsem.at[c],
                ).start()
                return v
            lax.fori_loop(0, CH, issue, 0)

        gather_chunk(0)

        barrier = pltpu.get_barrier_semaphore()
        pl.semaphore_signal(barrier, inc=1, device_id=(x, 1 - y),
                            device_id_type=pl.DeviceIdType.MESH)
        pl.semaphore_signal(barrier, inc=1, device_id=(1 - x, y),
                            device_id_type=pl.DeviceIdType.MESH)
        pl.semaphore_wait(barrier, 2)

        rdma_ys = [None] * C
        rdma_xs = [None] * C

        def y_step(c):
            pltpu.make_async_copy(
                E_ref.at[pl.ds(0, CH), :],
                gbuf.at[pl.ds(c * CH, CH), :],
                gsem.at[c],
            ).wait()
            sl = pl.ds(c * CH, CH)
            pbuf[sl, :] = (gbuf[sl, :] * m_ref[sl, :]).astype(jnp.bfloat16)
            r = pltpu.make_async_remote_copy(
                src_ref=pbuf.at[sl, :],
                dst_ref=bufy.at[sl, :],
                send_sem=ysend.at[c],
                recv_sem=yrecv.at[c],
                device_id=(x, 1 - y),
                device_id_type=pl.DeviceIdType.MESH,
            )
            r.start()
            rdma_ys[c] = r

        def x_step(c):
            rdma_ys[c].wait_recv()
            sl = pl.ds(c * CH, CH)
            row0 = x * TPX + c * CH
            out_ref[pl.ds(row0, CH), :] = pbuf[sl, :] + bufy[sl, :]
            r = pltpu.make_async_remote_copy(
                src_ref=out_ref.at[pl.ds(row0, CH), :],
                dst_ref=out_ref.at[pl.ds(row0, CH), :],
                send_sem=xsend.at[c],
                recv_sem=xrecv.at[c],
                device_id=(1 - x, y),
                device_id_type=pl.DeviceIdType.MESH,
            )
            r.start()
            rdma_xs[c] = r

        for c in range(C):
            y_step(c)
            if c + 1 < C:
                gather_chunk(c + 1)
            if c >= 1:
                x_step(c - 1)
        x_step(C - 1)

        for c in range(C):
            rdma_ys[c].wait_send()
            rdma_xs[c].wait_send()
            rdma_xs[c].wait_recv()

    grid_spec = pltpu.PrefetchScalarGridSpec(
        num_scalar_prefetch=1,
        in_specs=[
            pl.BlockSpec(memory_space=pl.ANY),
            pl.BlockSpec(memory_space=pltpu.VMEM),
        ],
        out_specs=pl.BlockSpec(memory_space=pltpu.VMEM),
        scratch_shapes=[
            pltpu.VMEM((TPX, D), jnp.float32),
            pltpu.VMEM((TPX, D), jnp.bfloat16),
            pltpu.VMEM((TPX, D), jnp.bfloat16),
            pltpu.SemaphoreType.DMA((C,)),
            pltpu.SemaphoreType.DMA((C,)),
            pltpu.SemaphoreType.DMA((C,)),
            pltpu.SemaphoreType.DMA((C,)),
            pltpu.SemaphoreType.DMA((C,)),
        ],
    )
    return pl.pallas_call(
        body,
        out_shape=jax.ShapeDtypeStruct((T, D), jnp.bfloat16),
        grid_spec=grid_spec,
        compiler_params=pltpu.CompilerParams(collective_id=0),
    )(idx, E, mask)
